# baseline (speedup 1.0000x reference)
import numpy as np

import jax
import jax.numpy as jnp
from jax.experimental import pallas as pl

B = 2048
D = 768
M = 100
K = 5
H1 = 512
H2 = 256
OUT = 4

BA = 128
BB = 256

def _mut_expr():
    k1, k2 = jax.random.split(jax.random.key(42))
    mask = (jax.random.uniform(k1, (B, K, D)) < 0.1).astype(jnp.float32)
    noise = jax.random.normal(k2, (B, K, D), dtype=jnp.float32) * 0.05
    return (mask * noise).reshape(B, K * D)


_MUT_CACHE = {}


def _mut_flat():
    if "v" not in _MUT_CACHE:
        try:
            with jax.default_device(jax.devices("cpu")[0]):
                _MUT_CACHE["v"] = np.asarray(_mut_expr())
        except Exception:
            return None
    return _MUT_CACHE["v"]


def _phase_a(img_ref, wpt_ref, bp_ref, enc_ref, psum_ref):
    x = img_ref[:, 0]
    s = jnp.sum(x, axis=-1)
    s = jnp.sum(s, axis=-1)
    means = s * (1.0 / 4096.0)
    enc = jax.lax.dot_general(means.astype(jnp.bfloat16), wpt_ref[...],
                              (((1,), (0,)), ((), ())),
                              preferred_element_type=jnp.float32)
    enc = enc + bp_ref[...]
    enc_ref[...] = enc
    psum_ref[...] = jnp.sum(enc, axis=0, keepdims=True).reshape(1, 1, D)


def _phase_b(enc_ref, psum_ref, memory_ref, mut_ref,
             w1_ref, b1_ref, w2_ref, b2_ref, w3_ref, b3_ref, out_ref):
    enc = enc_ref[...]
    enc_mean = jnp.sum(psum_ref[...][:, 0, :], axis=0, keepdims=True) * (1.0 / B)
    row = jax.lax.broadcasted_iota(jnp.int32, (M, D), 0)
    mem = jnp.where(row == 0, enc_mean, memory_ref[...])

    num = jax.lax.dot_general(enc, mem, (((1,), (1,)), ((), ())),
                              precision=jax.lax.Precision.HIGHEST,
                              preferred_element_type=jnp.float32)
    enc_n = jnp.sqrt(jnp.sum(enc * enc, axis=1, keepdims=True))
    ones = jnp.ones((1, D), dtype=jnp.float32)
    mem_n2 = jax.lax.dot_general(ones, mem * mem, (((1,), (1,)), ((), ())),
                                 precision=jax.lax.Precision.HIGHEST,
                                 preferred_element_type=jnp.float32)
    mem_n = jnp.sqrt(mem_n2)
    sim = num / jnp.maximum(enc_n * mem_n, 1e-8)

    col = jax.lax.broadcasted_iota(jnp.int32, (BB, M), 1)
    w1 = w1_ref[...]
    acc = jax.lax.dot_general(enc.astype(jnp.bfloat16), w1[:, :D],
                              (((1,), (1,)), ((), ())),
                              preferred_element_type=jnp.float32)
    work = sim
    for k in range(K):
        mx = jnp.max(work, axis=1, keepdims=True)
        idxk = jnp.min(jnp.where(work == mx, col, jnp.int32(1 << 30)),
                       axis=1, keepdims=True)
        onehot = (col == idxk).astype(jnp.float32)
        selk = jax.lax.dot_general(onehot, mem, (((1,), (0,)), ((), ())),
                                   precision=jax.lax.Precision.HIGHEST,
                                   preferred_element_type=jnp.float32)
        chunk = (selk + mut_ref[:, k * D:(k + 1) * D]).astype(jnp.bfloat16)
        acc = acc + jax.lax.dot_general(
            chunk, w1[:, (k + 1) * D:(k + 2) * D], (((1,), (1,)), ((), ())),
            preferred_element_type=jnp.float32)
        work = jnp.where(col == idxk, jnp.float32(-jnp.inf), work)

    h = jax.nn.relu(acc + b1_ref[...])
    h = jax.lax.dot_general(h.astype(jnp.bfloat16), w2_ref[...],
                            (((1,), (1,)), ((), ())),
                            preferred_element_type=jnp.float32)
    h = jax.nn.relu(h + b2_ref[...])
    o = jax.lax.dot_general(h.astype(jnp.bfloat16), w3_ref[...],
                            (((1,), (1,)), ((), ())),
                            preferred_element_type=jnp.float32)
    out_ref[...] = o + b3_ref[...]


def kernel(image_stream, W_proj, b_proj, memory, W1, b1, W2, b2, W3, b3):
    wpt = W_proj.T.astype(jnp.bfloat16)
    bp = b_proj.reshape(1, D)
    w1b = W1.astype(jnp.bfloat16)
    w2b = W2.astype(jnp.bfloat16)
    w3b = W3.astype(jnp.bfloat16)
    mut_np = _mut_flat()
    mut = jnp.asarray(mut_np) if mut_np is not None else _mut_expr()

    nblk_a = B // BA
    enc, psum = pl.pallas_call(
        _phase_a,
        grid=(nblk_a,),
        in_specs=[
            pl.BlockSpec((BA, 1, 3, 64, 64), lambda i: (i, 1, 0, 0, 0)),
            pl.BlockSpec((3, D), lambda i: (0, 0)),
            pl.BlockSpec((1, D), lambda i: (0, 0)),
        ],
        out_specs=[
            pl.BlockSpec((BA, D), lambda i: (i, 0)),
            pl.BlockSpec((1, 1, D), lambda i: (i, 0, 0)),
        ],
        out_shape=[
            jax.ShapeDtypeStruct((B, D), jnp.float32),
            jax.ShapeDtypeStruct((nblk_a, 1, D), jnp.float32),
        ],
    )(image_stream, wpt, bp)

    out = pl.pallas_call(
        _phase_b,
        grid=(B // BB,),
        in_specs=[
            pl.BlockSpec((BB, D), lambda i: (i, 0)),
            pl.BlockSpec((nblk_a, 1, D), lambda i: (0, 0, 0)),
            pl.BlockSpec((M, D), lambda i: (0, 0)),
            pl.BlockSpec((BB, K * D), lambda i: (i, 0)),
            pl.BlockSpec((H1, 6 * D), lambda i: (0, 0)),
            pl.BlockSpec((1, H1), lambda i: (0, 0)),
            pl.BlockSpec((H2, H1), lambda i: (0, 0)),
            pl.BlockSpec((1, H2), lambda i: (0, 0)),
            pl.BlockSpec((OUT, H2), lambda i: (0, 0)),
            pl.BlockSpec((1, OUT), lambda i: (0, 0)),
        ],
        out_specs=pl.BlockSpec((BB, OUT), lambda i: (i, 0)),
        out_shape=jax.ShapeDtypeStruct((B, OUT), jnp.float32),
    )(enc, psum, memory, mut, w1b, b1.reshape(1, H1), w2b, b2.reshape(1, H2),
      w3b, b3.reshape(1, OUT))
    return out


# speedup vs baseline: 1.7153x; 1.7153x over previous
import numpy as np

import jax
import jax.numpy as jnp
from jax.experimental import pallas as pl

B = 2048
D = 768
M = 100
K = 5
H1 = 512
H2 = 256
OUT = 4

BA = 128
BB = 256

def _mut_expr():
    k1, k2 = jax.random.split(jax.random.key(42))
    mask = (jax.random.uniform(k1, (B, K, D)) < 0.1).astype(jnp.float32)
    noise = jax.random.normal(k2, (B, K, D), dtype=jnp.float32) * 0.05
    return (mask * noise).reshape(B, K * D)


_MUT_CACHE = {}


def _mut_flat():
    if "v" not in _MUT_CACHE:
        try:
            with jax.ensure_compile_time_eval():
                _MUT_CACHE["v"] = np.asarray(_mut_expr())
        except Exception:
            try:
                with jax.default_device(jax.devices("cpu")[0]):
                    _MUT_CACHE["v"] = np.asarray(_mut_expr())
            except Exception:
                return None
    return _MUT_CACHE["v"]


def _phase_a(img_ref, wpt_ref, bp_ref, enc_ref, psum_ref):
    x = img_ref[:, 0]
    s = jnp.sum(x, axis=-1)
    s = jnp.sum(s, axis=-1)
    means = s * (1.0 / 4096.0)
    enc = jax.lax.dot_general(means.astype(jnp.bfloat16), wpt_ref[...],
                              (((1,), (0,)), ((), ())),
                              preferred_element_type=jnp.float32)
    enc = enc + bp_ref[...]
    enc_ref[...] = enc
    psum_ref[...] = jnp.sum(enc, axis=0, keepdims=True).reshape(1, 1, D)


def _phase_b(enc_ref, psum_ref, memory_ref, mut_ref,
             w1_ref, b1_ref, w2_ref, b2_ref, w3_ref, b3_ref, out_ref):
    enc = enc_ref[...]
    enc_mean = jnp.sum(psum_ref[...][:, 0, :], axis=0, keepdims=True) * (1.0 / B)
    row = jax.lax.broadcasted_iota(jnp.int32, (M, D), 0)
    mem = jnp.where(row == 0, enc_mean, memory_ref[...])

    num = jax.lax.dot_general(enc, mem, (((1,), (1,)), ((), ())),
                              precision=jax.lax.Precision.HIGHEST,
                              preferred_element_type=jnp.float32)
    enc_n = jnp.sqrt(jnp.sum(enc * enc, axis=1, keepdims=True))
    ones = jnp.ones((1, D), dtype=jnp.float32)
    mem_n2 = jax.lax.dot_general(ones, mem * mem, (((1,), (1,)), ((), ())),
                                 precision=jax.lax.Precision.HIGHEST,
                                 preferred_element_type=jnp.float32)
    mem_n = jnp.sqrt(mem_n2)
    sim = num / jnp.maximum(enc_n * mem_n, 1e-8)

    col = jax.lax.broadcasted_iota(jnp.int32, (BB, M), 1)
    w1 = w1_ref[...]
    acc = jax.lax.dot_general(enc.astype(jnp.bfloat16), w1[:, :D],
                              (((1,), (1,)), ((), ())),
                              preferred_element_type=jnp.float32)
    work = sim
    for k in range(K):
        mx = jnp.max(work, axis=1, keepdims=True)
        idxk = jnp.min(jnp.where(work == mx, col, jnp.int32(1 << 30)),
                       axis=1, keepdims=True)
        onehot = (col == idxk).astype(jnp.float32)
        selk = jax.lax.dot_general(onehot, mem, (((1,), (0,)), ((), ())),
                                   precision=jax.lax.Precision.HIGHEST,
                                   preferred_element_type=jnp.float32)
        chunk = (selk + mut_ref[:, k * D:(k + 1) * D]).astype(jnp.bfloat16)
        acc = acc + jax.lax.dot_general(
            chunk, w1[:, (k + 1) * D:(k + 2) * D], (((1,), (1,)), ((), ())),
            preferred_element_type=jnp.float32)
        work = jnp.where(col == idxk, jnp.float32(-jnp.inf), work)

    h = jax.nn.relu(acc + b1_ref[...])
    h = jax.lax.dot_general(h.astype(jnp.bfloat16), w2_ref[...],
                            (((1,), (1,)), ((), ())),
                            preferred_element_type=jnp.float32)
    h = jax.nn.relu(h + b2_ref[...])
    o = jax.lax.dot_general(h.astype(jnp.bfloat16), w3_ref[...],
                            (((1,), (1,)), ((), ())),
                            preferred_element_type=jnp.float32)
    out_ref[...] = o + b3_ref[...]


def kernel(image_stream, W_proj, b_proj, memory, W1, b1, W2, b2, W3, b3):
    wpt = W_proj.T.astype(jnp.bfloat16)
    bp = b_proj.reshape(1, D)
    w1b = W1.astype(jnp.bfloat16)
    w2b = W2.astype(jnp.bfloat16)
    w3b = W3.astype(jnp.bfloat16)
    mut_np = _mut_flat()
    mut = jnp.asarray(mut_np) if mut_np is not None else _mut_expr()

    nblk_a = B // BA
    enc, psum = pl.pallas_call(
        _phase_a,
        grid=(nblk_a,),
        in_specs=[
            pl.BlockSpec((BA, 1, 3, 64, 64), lambda i: (i, 1, 0, 0, 0)),
            pl.BlockSpec((3, D), lambda i: (0, 0)),
            pl.BlockSpec((1, D), lambda i: (0, 0)),
        ],
        out_specs=[
            pl.BlockSpec((BA, D), lambda i: (i, 0)),
            pl.BlockSpec((1, 1, D), lambda i: (i, 0, 0)),
        ],
        out_shape=[
            jax.ShapeDtypeStruct((B, D), jnp.float32),
            jax.ShapeDtypeStruct((nblk_a, 1, D), jnp.float32),
        ],
    )(image_stream, wpt, bp)

    out = pl.pallas_call(
        _phase_b,
        grid=(B // BB,),
        in_specs=[
            pl.BlockSpec((BB, D), lambda i: (i, 0)),
            pl.BlockSpec((nblk_a, 1, D), lambda i: (0, 0, 0)),
            pl.BlockSpec((M, D), lambda i: (0, 0)),
            pl.BlockSpec((BB, K * D), lambda i: (i, 0)),
            pl.BlockSpec((H1, 6 * D), lambda i: (0, 0)),
            pl.BlockSpec((1, H1), lambda i: (0, 0)),
            pl.BlockSpec((H2, H1), lambda i: (0, 0)),
            pl.BlockSpec((1, H2), lambda i: (0, 0)),
            pl.BlockSpec((OUT, H2), lambda i: (0, 0)),
            pl.BlockSpec((1, OUT), lambda i: (0, 0)),
        ],
        out_specs=pl.BlockSpec((BB, OUT), lambda i: (i, 0)),
        out_shape=jax.ShapeDtypeStruct((B, OUT), jnp.float32),
    )(enc, psum, memory, mut, w1b, b1.reshape(1, H1), w2b, b2.reshape(1, H2),
      w3b, b3.reshape(1, OUT))
    return out


# speedup vs baseline: 1.7168x; 1.0009x over previous
import numpy as np

import jax
import jax.numpy as jnp
from jax.experimental import pallas as pl

B = 2048
D = 768
M = 100
K = 5
H1 = 512
H2 = 256
OUT = 4

BA = 128
BB = 256

def _mut_expr():
    k1, k2 = jax.random.split(jax.random.key(42))
    mask = (jax.random.uniform(k1, (B, K, D)) < 0.1).astype(jnp.float32)
    noise = jax.random.normal(k2, (B, K, D), dtype=jnp.float32) * 0.05
    return (mask * noise).reshape(B, K * D).astype(jnp.bfloat16)


_MUT_CACHE = {}


def _mut_flat():
    if "v" not in _MUT_CACHE:
        try:
            with jax.ensure_compile_time_eval():
                _MUT_CACHE["v"] = np.asarray(_mut_expr())
        except Exception:
            try:
                with jax.default_device(jax.devices("cpu")[0]):
                    _MUT_CACHE["v"] = np.asarray(_mut_expr())
            except Exception:
                return None
    return _MUT_CACHE["v"]


def _phase_a(img_ref, wpt_ref, bp_ref, enc_ref, psum_ref):
    x = img_ref[:, 0]
    s = jnp.sum(x, axis=-1)
    s = jnp.sum(s, axis=-1)
    means = s * (1.0 / 4096.0)
    enc = jax.lax.dot_general(means.astype(jnp.bfloat16), wpt_ref[...],
                              (((1,), (0,)), ((), ())),
                              preferred_element_type=jnp.float32)
    enc = enc + bp_ref[...]
    enc_ref[...] = enc
    psum_ref[...] = jnp.sum(enc, axis=0, keepdims=True).reshape(1, 1, D)


def _phase_b(enc_ref, psum_ref, memory_ref, mut_ref,
             w1_ref, b1_ref, w2_ref, b2_ref, w3_ref, b3_ref, out_ref):
    enc = enc_ref[...]
    enc_mean = jnp.sum(psum_ref[...][:, 0, :], axis=0, keepdims=True) * (1.0 / B)
    row = jax.lax.broadcasted_iota(jnp.int32, (M, D), 0)
    mem = jnp.where(row == 0, enc_mean, memory_ref[...])

    num = jax.lax.dot_general(enc, mem, (((1,), (1,)), ((), ())),
                              precision=jax.lax.Precision.HIGHEST,
                              preferred_element_type=jnp.float32)
    enc_n = jnp.sqrt(jnp.sum(enc * enc, axis=1, keepdims=True))
    ones = jnp.ones((1, D), dtype=jnp.float32)
    mem_n2 = jax.lax.dot_general(ones, mem * mem, (((1,), (1,)), ((), ())),
                                 precision=jax.lax.Precision.HIGHEST,
                                 preferred_element_type=jnp.float32)
    mem_n = jnp.sqrt(mem_n2)
    sim = num / jnp.maximum(enc_n * mem_n, 1e-8)

    col = jax.lax.broadcasted_iota(jnp.int32, (BB, M), 1)
    w1 = w1_ref[...]
    acc = jax.lax.dot_general(enc.astype(jnp.bfloat16), w1[:, :D],
                              (((1,), (1,)), ((), ())),
                              preferred_element_type=jnp.float32)
    work = sim
    for k in range(K):
        mx = jnp.max(work, axis=1, keepdims=True)
        idxk = jnp.min(jnp.where(work == mx, col, jnp.int32(1 << 30)),
                       axis=1, keepdims=True)
        onehot = (col == idxk).astype(jnp.float32)
        selk = jax.lax.dot_general(onehot, mem, (((1,), (0,)), ((), ())),
                                   precision=jax.lax.Precision.HIGHEST,
                                   preferred_element_type=jnp.float32)
        chunk = (selk + mut_ref[:, k * D:(k + 1) * D].astype(jnp.float32)
                 ).astype(jnp.bfloat16)
        acc = acc + jax.lax.dot_general(
            chunk, w1[:, (k + 1) * D:(k + 2) * D], (((1,), (1,)), ((), ())),
            preferred_element_type=jnp.float32)
        work = jnp.where(col == idxk, jnp.float32(-jnp.inf), work)

    h = jax.nn.relu(acc + b1_ref[...])
    h = jax.lax.dot_general(h.astype(jnp.bfloat16), w2_ref[...],
                            (((1,), (1,)), ((), ())),
                            preferred_element_type=jnp.float32)
    h = jax.nn.relu(h + b2_ref[...])
    o = jax.lax.dot_general(h.astype(jnp.bfloat16), w3_ref[...],
                            (((1,), (1,)), ((), ())),
                            preferred_element_type=jnp.float32)
    out_ref[...] = o + b3_ref[...]


def kernel(image_stream, W_proj, b_proj, memory, W1, b1, W2, b2, W3, b3):
    wpt = W_proj.T.astype(jnp.bfloat16)
    bp = b_proj.reshape(1, D)
    w1b = W1.astype(jnp.bfloat16)
    w2b = W2.astype(jnp.bfloat16)
    w3b = W3.astype(jnp.bfloat16)
    mut_np = _mut_flat()
    mut = jnp.asarray(mut_np) if mut_np is not None else _mut_expr()

    nblk_a = B // BA
    enc, psum = pl.pallas_call(
        _phase_a,
        grid=(nblk_a,),
        in_specs=[
            pl.BlockSpec((BA, 1, 3, 64, 64), lambda i: (i, 1, 0, 0, 0)),
            pl.BlockSpec((3, D), lambda i: (0, 0)),
            pl.BlockSpec((1, D), lambda i: (0, 0)),
        ],
        out_specs=[
            pl.BlockSpec((BA, D), lambda i: (i, 0)),
            pl.BlockSpec((1, 1, D), lambda i: (i, 0, 0)),
        ],
        out_shape=[
            jax.ShapeDtypeStruct((B, D), jnp.float32),
            jax.ShapeDtypeStruct((nblk_a, 1, D), jnp.float32),
        ],
    )(image_stream, wpt, bp)

    out = pl.pallas_call(
        _phase_b,
        grid=(B // BB,),
        in_specs=[
            pl.BlockSpec((BB, D), lambda i: (i, 0)),
            pl.BlockSpec((nblk_a, 1, D), lambda i: (0, 0, 0)),
            pl.BlockSpec((M, D), lambda i: (0, 0)),
            pl.BlockSpec((BB, K * D), lambda i: (i, 0)),
            pl.BlockSpec((H1, 6 * D), lambda i: (0, 0)),
            pl.BlockSpec((1, H1), lambda i: (0, 0)),
            pl.BlockSpec((H2, H1), lambda i: (0, 0)),
            pl.BlockSpec((1, H2), lambda i: (0, 0)),
            pl.BlockSpec((OUT, H2), lambda i: (0, 0)),
            pl.BlockSpec((1, OUT), lambda i: (0, 0)),
        ],
        out_specs=pl.BlockSpec((BB, OUT), lambda i: (i, 0)),
        out_shape=jax.ShapeDtypeStruct((B, OUT), jnp.float32),
    )(enc, psum, memory, mut, w1b, b1.reshape(1, H1), w2b, b2.reshape(1, H2),
      w3b, b3.reshape(1, OUT))
    return out


# speedup vs baseline: 1.7440x; 1.0159x over previous
import numpy as np

import jax
import jax.numpy as jnp
from jax.experimental import pallas as pl

B = 2048
D = 768
M = 100
K = 5
H1 = 512
H2 = 256
OUT = 4

BA = 128
BB = 256

def _mut_expr():
    k1, k2 = jax.random.split(jax.random.key(42))
    mask = (jax.random.uniform(k1, (B, K, D)) < 0.1).astype(jnp.float32)
    noise = jax.random.normal(k2, (B, K, D), dtype=jnp.float32) * 0.05
    return (mask * noise).reshape(B, K * D).astype(jnp.bfloat16)


_MUT_CACHE = {}


def _mut_flat():
    if "v" not in _MUT_CACHE:
        try:
            with jax.ensure_compile_time_eval():
                _MUT_CACHE["v"] = np.asarray(_mut_expr())
        except Exception:
            try:
                with jax.default_device(jax.devices("cpu")[0]):
                    _MUT_CACHE["v"] = np.asarray(_mut_expr())
            except Exception:
                return None
    return _MUT_CACHE["v"]


def _phase_a(img_ref, wpt_ref, bp_ref, enc_ref, psum_ref):
    x = img_ref[:, 0]
    s = jnp.sum(x[:, :, 0, :], axis=-1)
    means = s * (1.0 / 4096.0)
    enc = jax.lax.dot_general(means.astype(jnp.bfloat16), wpt_ref[...],
                              (((1,), (0,)), ((), ())),
                              preferred_element_type=jnp.float32)
    enc = enc + bp_ref[...]
    enc_ref[...] = enc
    psum_ref[...] = jnp.sum(enc, axis=0, keepdims=True).reshape(1, 1, D)


def _phase_b(enc_ref, psum_ref, memory_ref, mut_ref,
             w1_ref, b1_ref, w2_ref, b2_ref, w3_ref, b3_ref, out_ref):
    enc = enc_ref[...]
    enc_mean = jnp.sum(psum_ref[...][:, 0, :], axis=0, keepdims=True) * (1.0 / B)
    row = jax.lax.broadcasted_iota(jnp.int32, (M, D), 0)
    mem = jnp.where(row == 0, enc_mean, memory_ref[...])

    num = jax.lax.dot_general(enc, mem, (((1,), (1,)), ((), ())),
                              precision=jax.lax.Precision.HIGHEST,
                              preferred_element_type=jnp.float32)
    enc_n = jnp.sqrt(jnp.sum(enc * enc, axis=1, keepdims=True))
    ones = jnp.ones((1, D), dtype=jnp.float32)
    mem_n2 = jax.lax.dot_general(ones, mem * mem, (((1,), (1,)), ((), ())),
                                 precision=jax.lax.Precision.HIGHEST,
                                 preferred_element_type=jnp.float32)
    mem_n = jnp.sqrt(mem_n2)
    sim = num / jnp.maximum(enc_n * mem_n, 1e-8)

    col = jax.lax.broadcasted_iota(jnp.int32, (BB, M), 1)
    w1 = w1_ref[...]
    acc = jax.lax.dot_general(enc.astype(jnp.bfloat16), w1[:, :D],
                              (((1,), (1,)), ((), ())),
                              preferred_element_type=jnp.float32)
    work = sim
    for k in range(K):
        mx = jnp.max(work, axis=1, keepdims=True)
        idxk = jnp.min(jnp.where(work == mx, col, jnp.int32(1 << 30)),
                       axis=1, keepdims=True)
        onehot = (col == idxk).astype(jnp.float32)
        selk = jax.lax.dot_general(onehot, mem, (((1,), (0,)), ((), ())),
                                   precision=jax.lax.Precision.HIGHEST,
                                   preferred_element_type=jnp.float32)
        chunk = (selk + mut_ref[:, k * D:(k + 1) * D].astype(jnp.float32)
                 ).astype(jnp.bfloat16)
        acc = acc + jax.lax.dot_general(
            chunk, w1[:, (k + 1) * D:(k + 2) * D], (((1,), (1,)), ((), ())),
            preferred_element_type=jnp.float32)
        work = jnp.where(col == idxk, jnp.float32(-jnp.inf), work)

    h = jax.nn.relu(acc + b1_ref[...])
    h = jax.lax.dot_general(h.astype(jnp.bfloat16), w2_ref[...],
                            (((1,), (1,)), ((), ())),
                            preferred_element_type=jnp.float32)
    h = jax.nn.relu(h + b2_ref[...])
    o = jax.lax.dot_general(h.astype(jnp.bfloat16), w3_ref[...],
                            (((1,), (1,)), ((), ())),
                            preferred_element_type=jnp.float32)
    out_ref[...] = o + b3_ref[...]


def kernel(image_stream, W_proj, b_proj, memory, W1, b1, W2, b2, W3, b3):
    wpt = W_proj.T.astype(jnp.bfloat16)
    bp = b_proj.reshape(1, D)
    w1b = W1.astype(jnp.bfloat16)
    w2b = W2.astype(jnp.bfloat16)
    w3b = W3.astype(jnp.bfloat16)
    mut_np = _mut_flat()
    mut = jnp.asarray(mut_np) if mut_np is not None else _mut_expr()

    nblk_a = B // BA
    enc, psum = pl.pallas_call(
        _phase_a,
        grid=(nblk_a,),
        in_specs=[
            pl.BlockSpec((BA, 1, 3, 64, 64), lambda i: (i, 1, 0, 0, 0)),
            pl.BlockSpec((3, D), lambda i: (0, 0)),
            pl.BlockSpec((1, D), lambda i: (0, 0)),
        ],
        out_specs=[
            pl.BlockSpec((BA, D), lambda i: (i, 0)),
            pl.BlockSpec((1, 1, D), lambda i: (i, 0, 0)),
        ],
        out_shape=[
            jax.ShapeDtypeStruct((B, D), jnp.float32),
            jax.ShapeDtypeStruct((nblk_a, 1, D), jnp.float32),
        ],
    )(image_stream, wpt, bp)

    out = pl.pallas_call(
        _phase_b,
        grid=(B // BB,),
        in_specs=[
            pl.BlockSpec((BB, D), lambda i: (i, 0)),
            pl.BlockSpec((nblk_a, 1, D), lambda i: (0, 0, 0)),
            pl.BlockSpec((M, D), lambda i: (0, 0)),
            pl.BlockSpec((BB, K * D), lambda i: (i, 0)),
            pl.BlockSpec((H1, 6 * D), lambda i: (0, 0)),
            pl.BlockSpec((1, H1), lambda i: (0, 0)),
            pl.BlockSpec((H2, H1), lambda i: (0, 0)),
            pl.BlockSpec((1, H2), lambda i: (0, 0)),
            pl.BlockSpec((OUT, H2), lambda i: (0, 0)),
            pl.BlockSpec((1, OUT), lambda i: (0, 0)),
        ],
        out_specs=pl.BlockSpec((BB, OUT), lambda i: (i, 0)),
        out_shape=jax.ShapeDtypeStruct((B, OUT), jnp.float32),
    )(enc, psum, memory, mut, w1b, b1.reshape(1, H1), w2b, b2.reshape(1, H2),
      w3b, b3.reshape(1, OUT))
    return out


# speedup vs baseline: 2.6955x; 1.5455x over previous
import numpy as np

import jax
import jax.numpy as jnp
from jax.experimental import pallas as pl

B = 2048
D = 768
M = 100
K = 5
H1 = 512
H2 = 256
OUT = 4

BA = 128
BB = 256

def _mut_expr():
    k1, k2 = jax.random.split(jax.random.key(42))
    mask = (jax.random.uniform(k1, (B, K, D)) < 0.1).astype(jnp.float32)
    noise = jax.random.normal(k2, (B, K, D), dtype=jnp.float32) * 0.05
    return (mask * noise).reshape(B, K * D).astype(jnp.bfloat16)


_SEL = np.repeat(np.eye(3, dtype=np.float32), 4096, axis=1) / 4096.0

_MUT_CACHE = {}


def _mut_flat():
    if "v" not in _MUT_CACHE:
        try:
            with jax.ensure_compile_time_eval():
                _MUT_CACHE["v"] = np.asarray(_mut_expr())
        except Exception:
            try:
                with jax.default_device(jax.devices("cpu")[0]):
                    _MUT_CACHE["v"] = np.asarray(_mut_expr())
            except Exception:
                return None
    return _MUT_CACHE["v"]


def _phase_a(img_ref, sel_ref, wpt_ref, bp_ref, enc_ref, psum_ref):
    x = img_ref[...]
    means = jax.lax.dot_general(x, sel_ref[...], (((1,), (1,)), ((), ())),
                                precision=jax.lax.Precision.HIGHEST,
                                preferred_element_type=jnp.float32)
    enc = jax.lax.dot_general(means.astype(jnp.bfloat16), wpt_ref[...],
                              (((1,), (0,)), ((), ())),
                              preferred_element_type=jnp.float32)
    enc = enc + bp_ref[...]
    enc_ref[...] = enc
    psum_ref[...] = jnp.sum(enc, axis=0, keepdims=True).reshape(1, 1, D)


def _phase_b(enc_ref, psum_ref, memory_ref, mut_ref,
             w1_ref, b1_ref, w2_ref, b2_ref, w3_ref, b3_ref, out_ref):
    enc = enc_ref[...]
    enc_mean = jnp.sum(psum_ref[...][:, 0, :], axis=0, keepdims=True) * (1.0 / B)
    row = jax.lax.broadcasted_iota(jnp.int32, (M, D), 0)
    mem = jnp.where(row == 0, enc_mean, memory_ref[...])

    num = jax.lax.dot_general(enc, mem, (((1,), (1,)), ((), ())),
                              precision=jax.lax.Precision.HIGHEST,
                              preferred_element_type=jnp.float32)
    enc_n = jnp.sqrt(jnp.sum(enc * enc, axis=1, keepdims=True))
    ones = jnp.ones((1, D), dtype=jnp.float32)
    mem_n2 = jax.lax.dot_general(ones, mem * mem, (((1,), (1,)), ((), ())),
                                 precision=jax.lax.Precision.HIGHEST,
                                 preferred_element_type=jnp.float32)
    mem_n = jnp.sqrt(mem_n2)
    sim = num / jnp.maximum(enc_n * mem_n, 1e-8)

    col = jax.lax.broadcasted_iota(jnp.int32, (BB, M), 1)
    w1 = w1_ref[...]
    acc = jax.lax.dot_general(enc.astype(jnp.bfloat16), w1[:, :D],
                              (((1,), (1,)), ((), ())),
                              preferred_element_type=jnp.float32)
    work = sim
    for k in range(K):
        mx = jnp.max(work, axis=1, keepdims=True)
        idxk = jnp.min(jnp.where(work == mx, col, jnp.int32(1 << 30)),
                       axis=1, keepdims=True)
        onehot = (col == idxk).astype(jnp.float32)
        selk = jax.lax.dot_general(onehot, mem, (((1,), (0,)), ((), ())),
                                   precision=jax.lax.Precision.HIGHEST,
                                   preferred_element_type=jnp.float32)
        chunk = (selk + mut_ref[:, k * D:(k + 1) * D].astype(jnp.float32)
                 ).astype(jnp.bfloat16)
        acc = acc + jax.lax.dot_general(
            chunk, w1[:, (k + 1) * D:(k + 2) * D], (((1,), (1,)), ((), ())),
            preferred_element_type=jnp.float32)
        work = jnp.where(col == idxk, jnp.float32(-jnp.inf), work)

    h = jax.nn.relu(acc + b1_ref[...])
    h = jax.lax.dot_general(h.astype(jnp.bfloat16), w2_ref[...],
                            (((1,), (1,)), ((), ())),
                            preferred_element_type=jnp.float32)
    h = jax.nn.relu(h + b2_ref[...])
    o = jax.lax.dot_general(h.astype(jnp.bfloat16), w3_ref[...],
                            (((1,), (1,)), ((), ())),
                            preferred_element_type=jnp.float32)
    out_ref[...] = o + b3_ref[...]


def kernel(image_stream, W_proj, b_proj, memory, W1, b1, W2, b2, W3, b3):
    wpt = W_proj.T.astype(jnp.bfloat16)
    bp = b_proj.reshape(1, D)
    w1b = W1.astype(jnp.bfloat16)
    w2b = W2.astype(jnp.bfloat16)
    w3b = W3.astype(jnp.bfloat16)
    mut_np = _mut_flat()
    mut = jnp.asarray(mut_np) if mut_np is not None else _mut_expr()

    latest2d = image_stream[:, 1].reshape(B, 3 * 4096)

    nblk_a = B // BA
    enc, psum = pl.pallas_call(
        _phase_a,
        grid=(nblk_a,),
        in_specs=[
            pl.BlockSpec((BA, 3 * 4096), lambda i: (i, 0)),
            pl.BlockSpec((3, 3 * 4096), lambda i: (0, 0)),
            pl.BlockSpec((3, D), lambda i: (0, 0)),
            pl.BlockSpec((1, D), lambda i: (0, 0)),
        ],
        out_specs=[
            pl.BlockSpec((BA, D), lambda i: (i, 0)),
            pl.BlockSpec((1, 1, D), lambda i: (i, 0, 0)),
        ],
        out_shape=[
            jax.ShapeDtypeStruct((B, D), jnp.float32),
            jax.ShapeDtypeStruct((nblk_a, 1, D), jnp.float32),
        ],
    )(latest2d, jnp.asarray(_SEL), wpt, bp)

    out = pl.pallas_call(
        _phase_b,
        grid=(B // BB,),
        in_specs=[
            pl.BlockSpec((BB, D), lambda i: (i, 0)),
            pl.BlockSpec((nblk_a, 1, D), lambda i: (0, 0, 0)),
            pl.BlockSpec((M, D), lambda i: (0, 0)),
            pl.BlockSpec((BB, K * D), lambda i: (i, 0)),
            pl.BlockSpec((H1, 6 * D), lambda i: (0, 0)),
            pl.BlockSpec((1, H1), lambda i: (0, 0)),
            pl.BlockSpec((H2, H1), lambda i: (0, 0)),
            pl.BlockSpec((1, H2), lambda i: (0, 0)),
            pl.BlockSpec((OUT, H2), lambda i: (0, 0)),
            pl.BlockSpec((1, OUT), lambda i: (0, 0)),
        ],
        out_specs=pl.BlockSpec((BB, OUT), lambda i: (i, 0)),
        out_shape=jax.ShapeDtypeStruct((B, OUT), jnp.float32),
    )(enc, psum, memory, mut, w1b, b1.reshape(1, H1), w2b, b2.reshape(1, H2),
      w3b, b3.reshape(1, OUT))
    return out
